# baseline (device time: 171950 ns/iter reference)
import jax
import jax.numpy as jnp
from jax import lax
from jax.experimental import pallas as pl
from jax.experimental.pallas import tpu as pltpu

N_DEV = 4
SUBS = 2


def kernel(x, w_mat):
    m_per, k = x.shape
    _, n_per = w_mat.shape
    half = m_per // 2
    sub = half // SUBS

    x = x.astype(jnp.bfloat16)
    w_mat = w_mat.astype(jnp.bfloat16)

    def body(x_ref, w_ref, out_ref, commR, commL,
             sendR, recvR, sendL, recvL):
        my = lax.axis_index("i")
        left = (my + N_DEV - 1) % N_DEV
        right = (my + 1) % N_DEV

        barrier_sem = pltpu.get_barrier_semaphore()
        for nbr in (left, right):
            pl.semaphore_signal(
                barrier_sem, inc=1,
                device_id=(nbr,), device_id_type=pl.DeviceIdType.MESH,
            )
        pl.semaphore_wait(barrier_sem, 2)

        def rcopy(src, dst, ssem, rsem, dev):
            return pltpu.make_async_remote_copy(
                src_ref=src, dst_ref=dst, send_sem=ssem, recv_sem=rsem,
                device_id=(dev,), device_id_type=pl.DeviceIdType.MESH,
            )

        r_fly, l_fly = [], []
        for s in range(SUBS):
            c = rcopy(
                x_ref.at[pl.ds(s * sub, sub), :],
                commR.at[0, pl.ds(s * sub, sub), :],
                sendR.at[0, s], recvR.at[0, s], right,
            )
            c.start()
            r_fly.append(c)
            c = rcopy(
                x_ref.at[pl.ds(half + s * sub, sub), :],
                commL.at[0, pl.ds(s * sub, sub), :],
                sendL.at[0, s], recvL.at[0, s], left,
            )
            c.start()
            l_fly.append(c)

        def gemm_store(src, origin, row_off, rows):
            out_ref[pl.ds(origin * m_per + row_off, rows), :] = jnp.maximum(
                jnp.dot(src, w_ref[...], preferred_element_type=jnp.float32),
                0.0,
            )

        gemm_store(x_ref[...], my, 0, m_per)

        for h in range(N_DEV - 2):
            r_next, l_next = [], []
            for s in range(SUBS):
                r_fly[s].wait()
                c = rcopy(
                    commR.at[h, pl.ds(s * sub, sub), :],
                    commR.at[h + 1, pl.ds(s * sub, sub), :],
                    sendR.at[h + 1, s], recvR.at[h + 1, s], right,
                )
                c.start()
                r_next.append(c)
                l_fly[s].wait()
                c = rcopy(
                    commL.at[h, pl.ds(s * sub, sub), :],
                    commL.at[h + 1, pl.ds(s * sub, sub), :],
                    sendL.at[h + 1, s], recvL.at[h + 1, s], left,
                )
                c.start()
                l_next.append(c)
            gemm_store(commR[h], (my + N_DEV - 1 - h) % N_DEV, 0, half)
            gemm_store(commL[h], (my + 1 + h) % N_DEV, half, half)
            r_fly, l_fly = r_next, l_next

        H = N_DEV - 2
        origin_r = (my + 1) % N_DEV
        origin_l = (my + N_DEV - 1) % N_DEV
        for s in range(SUBS):
            r_fly[s].wait()
            gemm_store(
                commR[H, pl.ds(s * sub, sub), :],
                origin_r, s * sub, sub,
            )
            l_fly[s].wait()
            gemm_store(
                commL[H, pl.ds(s * sub, sub), :],
                origin_l, half + s * sub, sub,
            )

    return pl.pallas_call(
        body,
        out_shape=jax.ShapeDtypeStruct((N_DEV * m_per, n_per), jnp.float32),
        in_specs=[
            pl.BlockSpec(memory_space=pltpu.VMEM),
            pl.BlockSpec(memory_space=pltpu.VMEM),
        ],
        out_specs=pl.BlockSpec(memory_space=pltpu.VMEM),
        scratch_shapes=[
            pltpu.VMEM((N_DEV - 1, half, k), jnp.bfloat16),
            pltpu.VMEM((N_DEV - 1, half, k), jnp.bfloat16),
            pltpu.SemaphoreType.DMA((N_DEV - 1, SUBS)),
            pltpu.SemaphoreType.DMA((N_DEV - 1, SUBS)),
            pltpu.SemaphoreType.DMA((N_DEV - 1, SUBS)),
            pltpu.SemaphoreType.DMA((N_DEV - 1, SUBS)),
        ],
        compiler_params=pltpu.CompilerParams(
            collective_id=0,
            vmem_limit_bytes=100 * 1024 * 1024,
        ),
    )(x, w_mat)


# device time: 158340 ns/iter; 1.0860x vs baseline; 1.0860x over previous
import jax
import jax.numpy as jnp
from jax import lax
from jax.experimental import pallas as pl
from jax.experimental.pallas import tpu as pltpu

N_DEV = 4


def kernel(x, w_mat):
    m_per, k = x.shape
    _, n_per = w_mat.shape

    x = x.astype(jnp.bfloat16)
    w_mat = w_mat.astype(jnp.bfloat16)

    half = m_per // 2
    SUBS = 4
    sub = half // SUBS

    def body(x_ref, w_ref, out_ref, commR, commL,
             sendR, recvR, sendL, recvL,
             sendR2, recvR2, sendL2, recvL2):
        my = lax.axis_index("i")
        left = (my + N_DEV - 1) % N_DEV
        right = (my + 1) % N_DEV

        barrier_sem = pltpu.get_barrier_semaphore()
        for nbr in (left, right):
            pl.semaphore_signal(
                barrier_sem, inc=1,
                device_id=(nbr,), device_id_type=pl.DeviceIdType.MESH,
            )
        pl.semaphore_wait(barrier_sem, 2)

        def gemm_store(src, origin, row_off):
            out_ref[pl.ds(origin * m_per + row_off, half), :] = jnp.maximum(
                jnp.dot(src, w_ref[...], preferred_element_type=jnp.float32),
                0.0,
            )

        rR = pltpu.make_async_remote_copy(
            src_ref=x_ref.at[pl.ds(0, half), :], dst_ref=commR.at[0],
            send_sem=sendR.at[0], recv_sem=recvR.at[0],
            device_id=(right,), device_id_type=pl.DeviceIdType.MESH,
        )
        rL = pltpu.make_async_remote_copy(
            src_ref=x_ref.at[pl.ds(half, half), :], dst_ref=commL.at[0],
            send_sem=sendL.at[0], recv_sem=recvL.at[0],
            device_id=(left,), device_id_type=pl.DeviceIdType.MESH,
        )
        rR.start()
        rL.start()

        out_ref[pl.ds(my * m_per, m_per), :] = jnp.maximum(
            jnp.dot(x_ref[...], w_ref[...], preferred_element_type=jnp.float32),
            0.0,
        )

        rR.wait()
        rL.wait()
        rR = pltpu.make_async_remote_copy(
            src_ref=commR.at[0], dst_ref=commR.at[1],
            send_sem=sendR.at[1], recv_sem=recvR.at[1],
            device_id=(right,), device_id_type=pl.DeviceIdType.MESH,
        )
        rL = pltpu.make_async_remote_copy(
            src_ref=commL.at[0], dst_ref=commL.at[1],
            send_sem=sendL.at[1], recv_sem=recvL.at[1],
            device_id=(left,), device_id_type=pl.DeviceIdType.MESH,
        )
        rR.start()
        rL.start()
        gemm_store(commR[0], (my + N_DEV - 1) % N_DEV, 0)
        gemm_store(commL[0], (my + 1) % N_DEV, half)

        rR.wait()
        rL.wait()
        subsR = []
        subsL = []
        for s in range(SUBS):
            sR = pltpu.make_async_remote_copy(
                src_ref=commR.at[1, pl.ds(s * sub, sub), :],
                dst_ref=commR.at[2, pl.ds(s * sub, sub), :],
                send_sem=sendR2.at[s], recv_sem=recvR2.at[s],
                device_id=(right,), device_id_type=pl.DeviceIdType.MESH,
            )
            sL = pltpu.make_async_remote_copy(
                src_ref=commL.at[1, pl.ds(s * sub, sub), :],
                dst_ref=commL.at[2, pl.ds(s * sub, sub), :],
                send_sem=sendL2.at[s], recv_sem=recvL2.at[s],
                device_id=(left,), device_id_type=pl.DeviceIdType.MESH,
            )
            sR.start()
            sL.start()
            subsR.append(sR)
            subsL.append(sL)

        gemm_store(commR[1], (my + N_DEV - 2) % N_DEV, 0)
        gemm_store(commL[1], (my + 2) % N_DEV, half)

        origin_r = (my + 1) % N_DEV
        origin_l = (my + N_DEV - 1) % N_DEV
        for s in range(SUBS):
            subsR[s].wait()
            out_ref[pl.ds(origin_r * m_per + s * sub, sub), :] = jnp.maximum(
                jnp.dot(
                    commR[2, pl.ds(s * sub, sub), :], w_ref[...],
                    preferred_element_type=jnp.float32,
                ),
                0.0,
            )
            subsL[s].wait()
            out_ref[pl.ds(origin_l * m_per + half + s * sub, sub), :] = (
                jnp.maximum(
                    jnp.dot(
                        commL[2, pl.ds(s * sub, sub), :], w_ref[...],
                        preferred_element_type=jnp.float32,
                    ),
                    0.0,
                )
            )

    return pl.pallas_call(
        body,
        out_shape=jax.ShapeDtypeStruct((N_DEV * m_per, n_per), jnp.float32),
        in_specs=[
            pl.BlockSpec(memory_space=pltpu.VMEM),
            pl.BlockSpec(memory_space=pltpu.VMEM),
        ],
        out_specs=pl.BlockSpec(memory_space=pltpu.VMEM),
        scratch_shapes=[
            pltpu.VMEM((N_DEV - 1, half, k), jnp.bfloat16),
            pltpu.VMEM((N_DEV - 1, half, k), jnp.bfloat16),
            pltpu.SemaphoreType.DMA((N_DEV - 1,)),
            pltpu.SemaphoreType.DMA((N_DEV - 1,)),
            pltpu.SemaphoreType.DMA((N_DEV - 1,)),
            pltpu.SemaphoreType.DMA((N_DEV - 1,)),
            pltpu.SemaphoreType.DMA((SUBS,)),
            pltpu.SemaphoreType.DMA((SUBS,)),
            pltpu.SemaphoreType.DMA((SUBS,)),
            pltpu.SemaphoreType.DMA((SUBS,)),
        ],
        compiler_params=pltpu.CompilerParams(collective_id=0),
    )(x, w_mat)
